# baseline (speedup 1.0000x reference)
import functools
import math

import jax
import jax.numpy as jnp
from jax.experimental import pallas as pl
from jax.experimental.pallas import tpu as pltpu

HEADS = 12
DIM_HEAD = 64
NUM_BUCKETS = 32
MAX_DISTANCE = 128
MASK_VALUE = -3.4028234663852886e38
MEM_SLOTS_PAD = 64


def _qkv_kernel(x_ref, w_ref, o_ref):
    o_ref[0] = jnp.dot(x_ref[...], w_ref[0],
                       preferred_element_type=jnp.float32
                       ).astype(jnp.bfloat16)


def _out_kernel(a_ref, w_ref, bias_ref, o_ref):
    hi = pl.program_id(0)
    part = jnp.dot(a_ref[0], w_ref[0], preferred_element_type=jnp.float32)

    @pl.when(hi == 0)
    def _():
        o_ref[...] = part + bias_ref[...]

    @pl.when(hi > 0)
    def _():
        o_ref[...] += part


def _bias_kernel(tab_ref, o_ref, *, bq):
    o_ref[...] = jnp.zeros_like(o_ref)
    sr = 128
    w = 3 * sr
    r = jax.lax.broadcasted_iota(jnp.int32, (sr, w), 0)
    cc = jax.lax.broadcasted_iota(jnp.int32, (sr, w), 1)
    delta = sr + r - cc
    npos = jnp.maximum(delta, 0)
    max_exact = NUM_BUCKETS // 2
    safe = jnp.maximum(npos, 1).astype(jnp.float32)
    val_large = max_exact + (
        jnp.log(safe * (1.0 / max_exact))
        * (max_exact / math.log(MAX_DISTANCE / max_exact))
    ).astype(jnp.int32)
    bucket = jnp.where(npos < max_exact, npos,
                       jnp.minimum(val_large, NUM_BUCKETS - 1))
    t31 = tab_ref[0, 0, NUM_BUCKETS - 1]
    adj = jnp.zeros((sr, w), jnp.float32)
    for t in range(NUM_BUCKETS - 1):
        adj = adj + jnp.where(bucket == t, tab_ref[0, 0, t] - t31, 0.0)
    adj = jnp.where(delta >= 0, adj, 0.0)
    for ri in range(bq // sr):
        cs = sr * (ri + 1)
        o_ref[0, ri * sr:(ri + 1) * sr, cs:cs + w] = adj


def _attn_kernel(q_ref, k_ref, v_ref, w_ref, km_ref, vm_ref, tab_ref,
                 mask_ref, gate_ref, o_ref, s_ref, *, bq, n, scale):
    qi = pl.program_id(1)
    q = q_ref[0]
    k = k_ref[0]
    v = v_ref[0]

    t31 = tab_ref[0, 0, NUM_BUCKETS - 1]
    s = jax.lax.dot_general(q, k, (((1,), (1,)), ((), ())),
                            preferred_element_type=jnp.float32) * scale + t31

    start = jnp.maximum(qi * (bq // 128) - 2, 0) * 128
    wstart = jnp.where(qi == 0, 2, 0) * 128
    s_ref[...] = s
    s_ref[:, pl.ds(start, 768)] += w_ref[0, :, pl.ds(wstart, 768)]
    s = s_ref[...]

    i = qi * bq + jax.lax.broadcasted_iota(jnp.int32, (bq, n), 0)
    j = jax.lax.broadcasted_iota(jnp.int32, (bq, n), 1)
    s = jnp.where(j <= i, s, MASK_VALUE)

    m = jnp.max(s, axis=1, keepdims=True)
    p = jnp.exp(s - m)
    l = jnp.sum(p, axis=1, keepdims=True)
    local = jnp.dot(p.astype(jnp.bfloat16), v,
                    preferred_element_type=jnp.float32) / l

    km = km_ref[0]
    vm = vm_ref[0]
    sm = jax.lax.dot_general(q, km, (((1,), (1,)), ((), ())),
                             preferred_element_type=jnp.float32) * scale
    mm = jnp.max(sm, axis=1, keepdims=True)
    pm = jnp.exp(sm - mm) * mask_ref[0, :, 0:MEM_SLOTS_PAD]
    lm = jnp.sum(pm, axis=1, keepdims=True)
    mem = jnp.dot(pm.astype(jnp.bfloat16), vm,
                  preferred_element_type=jnp.float32) / lm

    g = jax.nn.sigmoid(gate_ref[0, :, 0:1])
    o_ref[0] = (local * g + mem * (1.0 - g)).astype(jnp.bfloat16)


def kernel(x, k_mem, v_mem, mem_mask, Wq, Wkv, Wo, bo, null_k, null_v,
           gate_param, rel_bias_table):
    b, n, dim = x.shape
    h, d = HEADS, DIM_HEAD
    topk = k_mem.shape[2]
    scale = d ** -0.5
    rel_scale = d ** 0.5
    nc = 3 * h

    x2 = x.reshape(n, dim).astype(jnp.bfloat16)
    w3 = (jnp.concatenate([Wq, Wkv], axis=1)
          .reshape(dim, nc, d).transpose(1, 0, 2)
          .astype(jnp.bfloat16))

    qkv = pl.pallas_call(
        _qkv_kernel,
        grid=(nc,),
        in_specs=[
            pl.BlockSpec((n, dim), lambda c: (0, 0)),
            pl.BlockSpec((1, dim, d), lambda c: (c, 0, 0)),
        ],
        out_specs=pl.BlockSpec((1, n, d), lambda c: (c, 0, 0)),
        out_shape=jax.ShapeDtypeStruct((nc, n, d), jnp.bfloat16),
    )(x2, w3)

    km = jnp.concatenate([null_k, k_mem[0]], axis=1)
    vm = jnp.concatenate([null_v, v_mem[0]], axis=1)
    pad = MEM_SLOTS_PAD - (1 + topk)
    km = jnp.pad(km, ((0, 0), (0, pad), (0, 0))).astype(jnp.bfloat16)
    vm = jnp.pad(vm, ((0, 0), (0, pad), (0, 0))).astype(jnp.bfloat16)
    maskf = jnp.concatenate(
        [jnp.ones((h, 1), jnp.float32), mem_mask[0].astype(jnp.float32),
         jnp.zeros((h, pad), jnp.float32)], axis=1)
    maskp = jnp.pad(maskf, ((0, 0), (0, 128 - MEM_SLOTS_PAD)))
    maskp = maskp.reshape(h, 1, 128)
    tabp = jnp.pad(rel_bias_table.T * rel_scale,
                   ((0, 0), (0, 128 - NUM_BUCKETS))).reshape(h, 1, 128)
    gatep = jnp.broadcast_to(gate_param.reshape(h, 1, 1),
                             (h, 1, 128)).astype(jnp.float32)

    bq = 512
    wtile = pl.pallas_call(
        functools.partial(_bias_kernel, bq=bq),
        grid=(h,),
        in_specs=[pl.BlockSpec((1, 1, 128), lambda hi: (hi, 0, 0))],
        out_specs=pl.BlockSpec((1, bq, 1024), lambda hi: (hi, 0, 0)),
        out_shape=jax.ShapeDtypeStruct((h, bq, 1024), jnp.float32),
    )(tabp)

    attn = pl.pallas_call(
        functools.partial(_attn_kernel, bq=bq, n=n, scale=scale),
        grid=(h, n // bq),
        in_specs=[
            pl.BlockSpec((1, bq, d), lambda hi, qi: (hi, qi, 0)),
            pl.BlockSpec((1, n, d), lambda hi, qi: (h + hi, 0, 0)),
            pl.BlockSpec((1, n, d), lambda hi, qi: (2 * h + hi, 0, 0)),
            pl.BlockSpec((1, bq, 1024), lambda hi, qi: (hi, 0, 0)),
            pl.BlockSpec((1, MEM_SLOTS_PAD, d), lambda hi, qi: (hi, 0, 0)),
            pl.BlockSpec((1, MEM_SLOTS_PAD, d), lambda hi, qi: (hi, 0, 0)),
            pl.BlockSpec((1, 1, 128), lambda hi, qi: (hi, 0, 0)),
            pl.BlockSpec((1, 1, 128), lambda hi, qi: (hi, 0, 0)),
            pl.BlockSpec((1, 1, 128), lambda hi, qi: (hi, 0, 0)),
        ],
        out_specs=pl.BlockSpec((1, bq, d), lambda hi, qi: (hi, qi, 0)),
        out_shape=jax.ShapeDtypeStruct((h, n, d), jnp.bfloat16),
        scratch_shapes=[pltpu.VMEM((bq, n), jnp.float32)],
    )(qkv, qkv, qkv, wtile, km, vm, tabp, maskp, gatep)

    wo3 = Wo.reshape(h, d, dim).astype(jnp.bfloat16)
    bo2 = bo.reshape(1, dim)
    out = pl.pallas_call(
        _out_kernel,
        grid=(h,),
        in_specs=[
            pl.BlockSpec((1, n, d), lambda hi: (hi, 0, 0)),
            pl.BlockSpec((1, d, dim), lambda hi: (hi, 0, 0)),
            pl.BlockSpec((1, dim), lambda hi: (0, 0)),
        ],
        out_specs=pl.BlockSpec((n, dim), lambda hi: (0, 0)),
        out_shape=jax.ShapeDtypeStruct((n, dim), jnp.float32),
    )(attn, wo3, bo2)

    return out.reshape(b, n, dim)


# speedup vs baseline: 1.1356x; 1.1356x over previous
import functools
import math

import jax
import jax.numpy as jnp
from jax.experimental import pallas as pl
from jax.experimental.pallas import tpu as pltpu

HEADS = 12
DIM_HEAD = 64
NUM_BUCKETS = 32
MAX_DISTANCE = 128
MASK_VALUE = -3.4028234663852886e38
MEM_SLOTS_PAD = 64


def _qkv_kernel(x_ref, w_ref, o_ref):
    o_ref[0] = jnp.dot(x_ref[...], w_ref[0],
                       preferred_element_type=jnp.float32
                       ).astype(jnp.bfloat16)


def _out_kernel(a1_ref, a2_ref, w_ref, bias_ref, o_ref):
    hi = pl.program_id(0)
    nh = a1_ref.shape[1]
    p1 = jnp.dot(a1_ref[0], w_ref[0], preferred_element_type=jnp.float32)
    p2 = jnp.dot(a2_ref[0], w_ref[0], preferred_element_type=jnp.float32)

    @pl.when(hi == 0)
    def _():
        o_ref[0:nh, :] = p1 + bias_ref[...]
        o_ref[nh:, :] = p2 + bias_ref[...]

    @pl.when(hi > 0)
    def _():
        o_ref[0:nh, :] += p1
        o_ref[nh:, :] += p2


def _bias_kernel(tab_ref, o_ref, *, bq):
    o_ref[...] = jnp.zeros_like(o_ref)
    sr = 128
    w = 3 * sr
    r = jax.lax.broadcasted_iota(jnp.int32, (sr, w), 0)
    cc = jax.lax.broadcasted_iota(jnp.int32, (sr, w), 1)
    delta = 2 * sr + r - cc
    npos = jnp.maximum(delta, 0)
    max_exact = NUM_BUCKETS // 2
    safe = jnp.maximum(npos, 1).astype(jnp.float32)
    val_large = max_exact + (
        jnp.log(safe * (1.0 / max_exact))
        * (max_exact / math.log(MAX_DISTANCE / max_exact))
    ).astype(jnp.int32)
    bucket = jnp.where(npos < max_exact, npos,
                       jnp.minimum(val_large, NUM_BUCKETS - 1))
    t31 = tab_ref[0, 0, NUM_BUCKETS - 1]
    adj = jnp.zeros((sr, w), jnp.float32)
    for t in range(NUM_BUCKETS - 1):
        adj = adj + jnp.where(bucket == t, tab_ref[0, 0, t] - t31, 0.0)
    adj = jnp.where(delta >= 0, adj, 0.0)
    for ri in range(bq // sr):
        cs = bq + (ri - 2) * sr
        o_ref[0, ri * sr:(ri + 1) * sr, cs:cs + w] = adj


def _attn_kernel(q_ref, k_ref, v_ref, w_ref, km_ref, vm_ref, tab_ref,
                 mask_ref, gate_ref, o_ref, s_ref, *, bq, n, scale, qoff):
    qi = pl.program_id(1) + qoff
    q = q_ref[0]
    k = k_ref[0]
    v = v_ref[0]

    t31 = tab_ref[0, 0, NUM_BUCKETS - 1]
    s = jax.lax.dot_general(q, k, (((1,), (1,)), ((), ())),
                            preferred_element_type=jnp.float32) * scale + t31

    start = jnp.maximum(qi - 1, 0) * bq
    wstart = jnp.where(qi == 0, bq, 0)
    s_ref[...] = s
    s_ref[:, pl.ds(start, 2 * bq)] += w_ref[0, :, pl.ds(wstart, 2 * bq)]
    s = s_ref[...]

    i = qi * bq + jax.lax.broadcasted_iota(jnp.int32, (bq, n), 0)
    j = jax.lax.broadcasted_iota(jnp.int32, (bq, n), 1)
    s = jnp.where(j <= i, s, MASK_VALUE)

    m = jnp.max(s, axis=1, keepdims=True)
    p = jnp.exp(s - m)
    l = jnp.sum(p, axis=1, keepdims=True)
    local = jnp.dot(p.astype(jnp.bfloat16), v,
                    preferred_element_type=jnp.float32) / l

    km = km_ref[0]
    vm = vm_ref[0]
    sm = jax.lax.dot_general(q, km, (((1,), (1,)), ((), ())),
                             preferred_element_type=jnp.float32) * scale
    mm = jnp.max(sm, axis=1, keepdims=True)
    pm = jnp.exp(sm - mm) * mask_ref[0, :, 0:MEM_SLOTS_PAD]
    lm = jnp.sum(pm, axis=1, keepdims=True)
    mem = jnp.dot(pm.astype(jnp.bfloat16), vm,
                  preferred_element_type=jnp.float32) / lm

    g = jax.nn.sigmoid(gate_ref[0, :, 0:1])
    o_ref[0] = (local * g + mem * (1.0 - g)).astype(jnp.bfloat16)


def kernel(x, k_mem, v_mem, mem_mask, Wq, Wkv, Wo, bo, null_k, null_v,
           gate_param, rel_bias_table):
    b, n, dim = x.shape
    h, d = HEADS, DIM_HEAD
    topk = k_mem.shape[2]
    scale = d ** -0.5
    rel_scale = d ** 0.5
    nc = 3 * h

    x2 = x.reshape(n, dim).astype(jnp.bfloat16)
    w3 = (jnp.concatenate([Wq, Wkv], axis=1)
          .reshape(dim, nc, d).transpose(1, 0, 2)
          .astype(jnp.bfloat16))

    qkv = pl.pallas_call(
        _qkv_kernel,
        grid=(nc,),
        in_specs=[
            pl.BlockSpec((n, dim), lambda c: (0, 0)),
            pl.BlockSpec((1, dim, d), lambda c: (c, 0, 0)),
        ],
        out_specs=pl.BlockSpec((1, n, d), lambda c: (c, 0, 0)),
        out_shape=jax.ShapeDtypeStruct((nc, n, d), jnp.bfloat16),
    )(x2, w3)

    km = jnp.concatenate([null_k, k_mem[0]], axis=1)
    vm = jnp.concatenate([null_v, v_mem[0]], axis=1)
    pad = MEM_SLOTS_PAD - (1 + topk)
    km = jnp.pad(km, ((0, 0), (0, pad), (0, 0))).astype(jnp.bfloat16)
    vm = jnp.pad(vm, ((0, 0), (0, pad), (0, 0))).astype(jnp.bfloat16)
    maskf = jnp.concatenate(
        [jnp.ones((h, 1), jnp.float32), mem_mask[0].astype(jnp.float32),
         jnp.zeros((h, pad), jnp.float32)], axis=1)
    maskp = jnp.pad(maskf, ((0, 0), (0, 128 - MEM_SLOTS_PAD)))
    maskp = maskp.reshape(h, 1, 128)
    tabp = jnp.pad(rel_bias_table.T * rel_scale,
                   ((0, 0), (0, 128 - NUM_BUCKETS))).reshape(h, 1, 128)
    gatep = jnp.broadcast_to(gate_param.reshape(h, 1, 1),
                             (h, 1, 128)).astype(jnp.float32)

    bq = 256
    wtile = pl.pallas_call(
        functools.partial(_bias_kernel, bq=bq),
        grid=(h,),
        in_specs=[pl.BlockSpec((1, 1, 128), lambda hi: (hi, 0, 0))],
        out_specs=pl.BlockSpec((1, bq, 3 * bq), lambda hi: (hi, 0, 0)),
        out_shape=jax.ShapeDtypeStruct((h, bq, 3 * bq), jnp.float32),
    )(tabp)

    def attn_call(n_eff, qoff, nq):
        return pl.pallas_call(
            functools.partial(_attn_kernel, bq=bq, n=n_eff, scale=scale,
                              qoff=qoff),
            grid=(h, nq),
            in_specs=[
                pl.BlockSpec((1, bq, d),
                             lambda hi, qi: (hi, qi + qoff, 0)),
                pl.BlockSpec((1, n_eff, d), lambda hi, qi: (h + hi, 0, 0)),
                pl.BlockSpec((1, n_eff, d),
                             lambda hi, qi: (2 * h + hi, 0, 0)),
                pl.BlockSpec((1, bq, 3 * bq), lambda hi, qi: (hi, 0, 0)),
                pl.BlockSpec((1, MEM_SLOTS_PAD, d),
                             lambda hi, qi: (hi, 0, 0)),
                pl.BlockSpec((1, MEM_SLOTS_PAD, d),
                             lambda hi, qi: (hi, 0, 0)),
                pl.BlockSpec((1, 1, 128), lambda hi, qi: (hi, 0, 0)),
                pl.BlockSpec((1, 1, 128), lambda hi, qi: (hi, 0, 0)),
                pl.BlockSpec((1, 1, 128), lambda hi, qi: (hi, 0, 0)),
            ],
            out_specs=pl.BlockSpec((1, bq, d), lambda hi, qi: (hi, qi, 0)),
            out_shape=jax.ShapeDtypeStruct((h, nq * bq, d), jnp.bfloat16),
            scratch_shapes=[pltpu.VMEM((bq, n_eff), jnp.float32)],
        )(qkv, qkv, qkv, wtile, km, vm, tabp, maskp, gatep)

    nh = n // 2
    attn_lo = attn_call(nh, 0, nh // bq)
    attn_hi = attn_call(n, nh // bq, nh // bq)

    wo3 = Wo.reshape(h, d, dim).astype(jnp.bfloat16)
    bo2 = bo.reshape(1, dim)
    out = pl.pallas_call(
        _out_kernel,
        grid=(h,),
        in_specs=[
            pl.BlockSpec((1, nh, d), lambda hi: (hi, 0, 0)),
            pl.BlockSpec((1, nh, d), lambda hi: (hi, 0, 0)),
            pl.BlockSpec((1, d, dim), lambda hi: (hi, 0, 0)),
            pl.BlockSpec((1, dim), lambda hi: (0, 0)),
        ],
        out_specs=pl.BlockSpec((n, dim), lambda hi: (0, 0)),
        out_shape=jax.ShapeDtypeStruct((n, dim), jnp.float32),
    )(attn_lo, attn_hi, wo3, bo2)

    return out.reshape(b, n, dim)


# speedup vs baseline: 1.2063x; 1.0622x over previous
import functools
import math

import jax
import jax.numpy as jnp
from jax.experimental import pallas as pl
from jax.experimental.pallas import tpu as pltpu

HEADS = 12
DIM_HEAD = 64
NUM_BUCKETS = 32
MAX_DISTANCE = 128
MASK_VALUE = -3.4028234663852886e38
MEM_SLOTS_PAD = 64


def _qkv_kernel(x_ref, w_ref, o_ref, *, scale):
    c = pl.program_id(0)
    sc = jnp.where(c < HEADS, scale, 1.0)
    o_ref[0] = (jnp.dot(x_ref[...], w_ref[0],
                        preferred_element_type=jnp.float32)
                * sc).astype(jnp.bfloat16)


def _out_kernel(a1_ref, a2_ref, w_ref, bias_ref, o_ref):
    nh = a1_ref.shape[1]
    a1 = jnp.concatenate([a1_ref[hh] for hh in range(HEADS)], axis=1)
    a2 = jnp.concatenate([a2_ref[hh] for hh in range(HEADS)], axis=1)
    o_ref[0:nh, :] = (jnp.dot(a1, w_ref[...],
                              preferred_element_type=jnp.float32)
                      + bias_ref[...])
    o_ref[nh:, :] = (jnp.dot(a2, w_ref[...],
                             preferred_element_type=jnp.float32)
                     + bias_ref[...])


def _bias_kernel(tab_ref, o_ref, *, bq):
    o_ref[...] = jnp.zeros_like(o_ref)
    sr = 128
    w = 3 * sr
    r = jax.lax.broadcasted_iota(jnp.int32, (sr, w), 0)
    cc = jax.lax.broadcasted_iota(jnp.int32, (sr, w), 1)
    delta = 2 * sr + r - cc
    npos = jnp.maximum(delta, 0)
    max_exact = NUM_BUCKETS // 2
    safe = jnp.maximum(npos, 1).astype(jnp.float32)
    val_large = max_exact + (
        jnp.log(safe * (1.0 / max_exact))
        * (max_exact / math.log(MAX_DISTANCE / max_exact))
    ).astype(jnp.int32)
    bucket = jnp.where(npos < max_exact, npos,
                       jnp.minimum(val_large, NUM_BUCKETS - 1))
    t31 = tab_ref[0, 0, NUM_BUCKETS - 1]
    adj = jnp.zeros((sr, w), jnp.float32)
    for t in range(NUM_BUCKETS - 1):
        adj = adj + jnp.where(bucket == t, tab_ref[0, 0, t] - t31, 0.0)
    adj = jnp.where(delta >= 0, adj, 0.0)
    for ri in range(bq // sr):
        cs = bq + (ri - 2) * sr
        o_ref[0, ri * sr:(ri + 1) * sr, cs:cs + w] = adj


def _attn_kernel(q_ref, k_ref, v_ref, w_ref, km_ref, vm_ref, tab_ref,
                 mask_ref, gate_ref, o_ref, s_ref, *, bq, n, scale, qoff):
    qi = pl.program_id(1) + qoff
    q = q_ref[0]
    k = k_ref[0]
    v = v_ref[0]

    s = jax.lax.dot_general(q, k, (((1,), (1,)), ((), ())),
                            preferred_element_type=jnp.float32)

    start = jnp.maximum(qi - 1, 0) * bq
    wstart = jnp.where(qi == 0, bq, 0)
    s_ref[...] = s
    s_ref[:, pl.ds(start, 2 * bq)] += w_ref[0, :, pl.ds(wstart, 2 * bq)]
    s = s_ref[...]

    i = qi * bq + jax.lax.broadcasted_iota(jnp.int32, (bq, n), 0)
    j = jax.lax.broadcasted_iota(jnp.int32, (bq, n), 1)
    s = jnp.where(j <= i, s, MASK_VALUE)

    m = jnp.max(s, axis=1, keepdims=True)
    p = jnp.exp(s - m)
    l = jnp.sum(p, axis=1, keepdims=True)
    local = jnp.dot(p.astype(jnp.bfloat16), v,
                    preferred_element_type=jnp.float32) / l

    km = km_ref[0]
    vm = vm_ref[0]
    sm = jax.lax.dot_general(q, km, (((1,), (1,)), ((), ())),
                             preferred_element_type=jnp.float32)
    mm = jnp.max(sm, axis=1, keepdims=True)
    pm = jnp.exp(sm - mm) * mask_ref[0, :, 0:MEM_SLOTS_PAD]
    lm = jnp.sum(pm, axis=1, keepdims=True)
    mem = jnp.dot(pm.astype(jnp.bfloat16), vm,
                  preferred_element_type=jnp.float32) / lm

    g = jax.nn.sigmoid(gate_ref[0, :, 0:1])
    o_ref[0] = (local * g + mem * (1.0 - g)).astype(jnp.bfloat16)


def kernel(x, k_mem, v_mem, mem_mask, Wq, Wkv, Wo, bo, null_k, null_v,
           gate_param, rel_bias_table):
    b, n, dim = x.shape
    h, d = HEADS, DIM_HEAD
    topk = k_mem.shape[2]
    scale = d ** -0.5
    rel_scale = d ** 0.5
    nc = 3 * h

    x2 = x.reshape(n, dim).astype(jnp.bfloat16)
    w3 = (jnp.concatenate([Wq, Wkv], axis=1)
          .reshape(dim, nc, d).transpose(1, 0, 2)
          .astype(jnp.bfloat16))

    qkv = pl.pallas_call(
        functools.partial(_qkv_kernel, scale=scale),
        grid=(nc,),
        in_specs=[
            pl.BlockSpec((n, dim), lambda c: (0, 0)),
            pl.BlockSpec((1, dim, d), lambda c: (c, 0, 0)),
        ],
        out_specs=pl.BlockSpec((1, n, d), lambda c: (c, 0, 0)),
        out_shape=jax.ShapeDtypeStruct((nc, n, d), jnp.bfloat16),
    )(x2, w3)

    km = jnp.concatenate([null_k, k_mem[0]], axis=1)
    vm = jnp.concatenate([null_v, v_mem[0]], axis=1)
    pad = MEM_SLOTS_PAD - (1 + topk)
    km = jnp.pad(km, ((0, 0), (0, pad), (0, 0))).astype(jnp.bfloat16)
    vm = jnp.pad(vm, ((0, 0), (0, pad), (0, 0))).astype(jnp.bfloat16)
    maskf = jnp.concatenate(
        [jnp.ones((h, 1), jnp.float32), mem_mask[0].astype(jnp.float32),
         jnp.zeros((h, pad), jnp.float32)], axis=1)
    maskp = jnp.pad(maskf, ((0, 0), (0, 128 - MEM_SLOTS_PAD)))
    maskp = maskp.reshape(h, 1, 128)
    tabp = jnp.pad(rel_bias_table.T * rel_scale,
                   ((0, 0), (0, 128 - NUM_BUCKETS))).reshape(h, 1, 128)
    gatep = jnp.broadcast_to(gate_param.reshape(h, 1, 1),
                             (h, 1, 128)).astype(jnp.float32)

    bq = 256
    wtile = pl.pallas_call(
        functools.partial(_bias_kernel, bq=bq),
        grid=(h,),
        in_specs=[pl.BlockSpec((1, 1, 128), lambda hi: (hi, 0, 0))],
        out_specs=pl.BlockSpec((1, bq, 3 * bq), lambda hi: (hi, 0, 0)),
        out_shape=jax.ShapeDtypeStruct((h, bq, 3 * bq), jnp.float32),
    )(tabp)

    def attn_call(n_eff, qoff, nq):
        return pl.pallas_call(
            functools.partial(_attn_kernel, bq=bq, n=n_eff, scale=scale,
                              qoff=qoff),
            grid=(h, nq),
            in_specs=[
                pl.BlockSpec((1, bq, d),
                             lambda hi, qi: (hi, qi + qoff, 0)),
                pl.BlockSpec((1, n_eff, d), lambda hi, qi: (h + hi, 0, 0)),
                pl.BlockSpec((1, n_eff, d),
                             lambda hi, qi: (2 * h + hi, 0, 0)),
                pl.BlockSpec((1, bq, 3 * bq), lambda hi, qi: (hi, 0, 0)),
                pl.BlockSpec((1, MEM_SLOTS_PAD, d),
                             lambda hi, qi: (hi, 0, 0)),
                pl.BlockSpec((1, MEM_SLOTS_PAD, d),
                             lambda hi, qi: (hi, 0, 0)),
                pl.BlockSpec((1, 1, 128), lambda hi, qi: (hi, 0, 0)),
                pl.BlockSpec((1, 1, 128), lambda hi, qi: (hi, 0, 0)),
                pl.BlockSpec((1, 1, 128), lambda hi, qi: (hi, 0, 0)),
            ],
            out_specs=pl.BlockSpec((1, bq, d), lambda hi, qi: (hi, qi, 0)),
            out_shape=jax.ShapeDtypeStruct((h, nq * bq, d), jnp.bfloat16),
            scratch_shapes=[pltpu.VMEM((bq, n_eff), jnp.float32)],
        )(qkv, qkv, qkv, wtile, km, vm, tabp, maskp, gatep)

    nh = n // 2
    attn_lo = attn_call(nh, 0, nh // bq)
    attn_hi = attn_call(n, nh // bq, nh // bq)

    wo2 = Wo.astype(jnp.bfloat16)
    bo2 = bo.reshape(1, dim)
    out = pl.pallas_call(
        _out_kernel,
        grid=(1,),
        in_specs=[
            pl.BlockSpec((h, nh, d), lambda i: (0, 0, 0)),
            pl.BlockSpec((h, nh, d), lambda i: (0, 0, 0)),
            pl.BlockSpec((h * d, dim), lambda i: (0, 0)),
            pl.BlockSpec((1, dim), lambda i: (0, 0)),
        ],
        out_specs=pl.BlockSpec((n, dim), lambda i: (0, 0)),
        out_shape=jax.ShapeDtypeStruct((n, dim), jnp.float32),
    )(attn_lo, attn_hi, wo2, bo2)

    return out.reshape(b, n, dim)


# speedup vs baseline: 1.2578x; 1.0427x over previous
import functools
import math

import jax
import jax.numpy as jnp
from jax.experimental import pallas as pl
from jax.experimental.pallas import tpu as pltpu

HEADS = 12
DIM_HEAD = 64
NUM_BUCKETS = 32
MAX_DISTANCE = 128
MASK_VALUE = -3.4028234663852886e38
MEM_SLOTS_PAD = 64


def _qkv_kernel(x_ref, w_ref, o_ref, *, scale):
    c = pl.program_id(0)
    sc = jnp.where(c < HEADS, scale, 1.0)
    o_ref[0] = (jnp.dot(x_ref[...], w_ref[0],
                        preferred_element_type=jnp.float32)
                * sc).astype(jnp.bfloat16)


def _out_kernel(a1_ref, a2_ref, w_ref, bias_ref, o_ref):
    nh = a1_ref.shape[1]
    a1 = jnp.concatenate([a1_ref[hh] for hh in range(HEADS)], axis=1)
    a2 = jnp.concatenate([a2_ref[hh] for hh in range(HEADS)], axis=1)
    o_ref[0:nh, :] = (jnp.dot(a1, w_ref[...],
                              preferred_element_type=jnp.float32)
                      + bias_ref[...])
    o_ref[nh:, :] = (jnp.dot(a2, w_ref[...],
                             preferred_element_type=jnp.float32)
                     + bias_ref[...])


def _bias_kernel(tab_ref, o_ref, *, bq):
    o_ref[...] = jnp.zeros_like(o_ref)
    sr = 128
    w = 3 * sr
    r = jax.lax.broadcasted_iota(jnp.int32, (sr, w), 0)
    cc = jax.lax.broadcasted_iota(jnp.int32, (sr, w), 1)
    delta = 2 * sr + r - cc
    npos = jnp.maximum(delta, 0)
    max_exact = NUM_BUCKETS // 2
    safe = jnp.maximum(npos, 1).astype(jnp.float32)
    val_large = max_exact + (
        jnp.log(safe * (1.0 / max_exact))
        * (max_exact / math.log(MAX_DISTANCE / max_exact))
    ).astype(jnp.int32)
    bucket = jnp.where(npos < max_exact, npos,
                       jnp.minimum(val_large, NUM_BUCKETS - 1))
    t31 = tab_ref[0, 0, NUM_BUCKETS - 1]
    adj = jnp.zeros((sr, w), jnp.float32)
    for t in range(NUM_BUCKETS - 1):
        adj = adj + jnp.where(bucket == t, tab_ref[0, 0, t] - t31, 0.0)
    adj = jnp.where(delta >= 0, adj, 0.0)
    for ri in range(bq // sr):
        cs = bq + (ri - 2) * sr
        o_ref[0, ri * sr:(ri + 1) * sr, cs:cs + w] = adj


def _attn_kernel(q_ref, k_ref, v_ref, w_ref, km_ref, vm_ref, tab_ref,
                 mask_ref, gate_ref, o_ref, s_ref, *, bq, n, scale, qoff):
    qi = pl.program_id(1) + qoff
    start = jnp.maximum(qi - 1, 0) * bq
    wstart = jnp.where(qi == 0, bq, 0)
    i = qi * bq + jax.lax.broadcasted_iota(jnp.int32, (bq, n), 0)
    j = jax.lax.broadcasted_iota(jnp.int32, (bq, n), 1)
    valid = j <= i

    for hh in range(2):
        q = q_ref[hh]
        k = k_ref[hh]
        v = v_ref[hh]

        s = jax.lax.dot_general(q, k, (((1,), (1,)), ((), ())),
                                preferred_element_type=jnp.float32)

        s_ref[hh] = s
        s_ref[hh, :, pl.ds(start, 2 * bq)] += \
            w_ref[hh, :, pl.ds(wstart, 2 * bq)]
        s = jnp.where(valid, s_ref[hh], MASK_VALUE)

        m = jnp.max(s, axis=1, keepdims=True)
        p = jnp.exp(s - m)
        l = jnp.sum(p, axis=1, keepdims=True)
        local = jnp.dot(p.astype(jnp.bfloat16), v,
                        preferred_element_type=jnp.float32) / l

        km = km_ref[hh]
        vm = vm_ref[hh]
        sm = jax.lax.dot_general(q, km, (((1,), (1,)), ((), ())),
                                 preferred_element_type=jnp.float32)
        mm = jnp.max(sm, axis=1, keepdims=True)
        pm = jnp.exp(sm - mm) * mask_ref[hh, :, 0:MEM_SLOTS_PAD]
        lm = jnp.sum(pm, axis=1, keepdims=True)
        mem = jnp.dot(pm.astype(jnp.bfloat16), vm,
                      preferred_element_type=jnp.float32) / lm

        g = jax.nn.sigmoid(gate_ref[hh, :, 0:1])
        o_ref[hh] = (local * g + mem * (1.0 - g)).astype(jnp.bfloat16)


def kernel(x, k_mem, v_mem, mem_mask, Wq, Wkv, Wo, bo, null_k, null_v,
           gate_param, rel_bias_table):
    b, n, dim = x.shape
    h, d = HEADS, DIM_HEAD
    topk = k_mem.shape[2]
    scale = d ** -0.5
    rel_scale = d ** 0.5
    nc = 3 * h

    x2 = x.reshape(n, dim).astype(jnp.bfloat16)
    w3 = (jnp.concatenate([Wq, Wkv], axis=1)
          .reshape(dim, nc, d).transpose(1, 0, 2)
          .astype(jnp.bfloat16))

    qkv = pl.pallas_call(
        functools.partial(_qkv_kernel, scale=scale),
        grid=(nc,),
        in_specs=[
            pl.BlockSpec((n, dim), lambda c: (0, 0)),
            pl.BlockSpec((1, dim, d), lambda c: (c, 0, 0)),
        ],
        out_specs=pl.BlockSpec((1, n, d), lambda c: (c, 0, 0)),
        out_shape=jax.ShapeDtypeStruct((nc, n, d), jnp.bfloat16),
    )(x2, w3)

    km = jnp.concatenate([null_k, k_mem[0]], axis=1)
    vm = jnp.concatenate([null_v, v_mem[0]], axis=1)
    pad = MEM_SLOTS_PAD - (1 + topk)
    km = jnp.pad(km, ((0, 0), (0, pad), (0, 0))).astype(jnp.bfloat16)
    vm = jnp.pad(vm, ((0, 0), (0, pad), (0, 0))).astype(jnp.bfloat16)
    maskf = jnp.concatenate(
        [jnp.ones((h, 1), jnp.float32), mem_mask[0].astype(jnp.float32),
         jnp.zeros((h, pad), jnp.float32)], axis=1)
    maskp = jnp.pad(maskf, ((0, 0), (0, 128 - MEM_SLOTS_PAD)))
    maskp = maskp.reshape(h, 1, 128)
    tabp = jnp.pad(rel_bias_table.T * rel_scale,
                   ((0, 0), (0, 128 - NUM_BUCKETS))).reshape(h, 1, 128)
    gatep = jnp.broadcast_to(gate_param.reshape(h, 1, 1),
                             (h, 1, 128)).astype(jnp.float32)

    bq = 256
    wtile = pl.pallas_call(
        functools.partial(_bias_kernel, bq=bq),
        grid=(h,),
        in_specs=[pl.BlockSpec((1, 1, 128), lambda hi: (hi, 0, 0))],
        out_specs=pl.BlockSpec((1, bq, 3 * bq), lambda hi: (hi, 0, 0)),
        out_shape=jax.ShapeDtypeStruct((h, bq, 3 * bq), jnp.float32),
    )(tabp)

    def attn_call(n_eff, qoff, nq):
        return pl.pallas_call(
            functools.partial(_attn_kernel, bq=bq, n=n_eff, scale=scale,
                              qoff=qoff),
            grid=(h // 2, nq),
            in_specs=[
                pl.BlockSpec((2, bq, d),
                             lambda hi, qi: (hi, qi + qoff, 0)),
                pl.BlockSpec((2, n_eff, d),
                             lambda hi, qi: (h // 2 + hi, 0, 0)),
                pl.BlockSpec((2, n_eff, d),
                             lambda hi, qi: (h + hi, 0, 0)),
                pl.BlockSpec((2, bq, 3 * bq), lambda hi, qi: (hi, 0, 0)),
                pl.BlockSpec((2, MEM_SLOTS_PAD, d),
                             lambda hi, qi: (hi, 0, 0)),
                pl.BlockSpec((2, MEM_SLOTS_PAD, d),
                             lambda hi, qi: (hi, 0, 0)),
                pl.BlockSpec((2, 1, 128), lambda hi, qi: (hi, 0, 0)),
                pl.BlockSpec((2, 1, 128), lambda hi, qi: (hi, 0, 0)),
                pl.BlockSpec((2, 1, 128), lambda hi, qi: (hi, 0, 0)),
            ],
            out_specs=pl.BlockSpec((2, bq, d), lambda hi, qi: (hi, qi, 0)),
            out_shape=jax.ShapeDtypeStruct((h, nq * bq, d), jnp.bfloat16),
            scratch_shapes=[pltpu.VMEM((2, bq, n_eff), jnp.float32)],
        )(qkv, qkv, qkv, wtile, km, vm, tabp, maskp, gatep)

    nh = n // 2
    attn_lo = attn_call(nh, 0, nh // bq)
    attn_hi = attn_call(n, nh // bq, nh // bq)

    wo2 = Wo.astype(jnp.bfloat16)
    bo2 = bo.reshape(1, dim)
    out = pl.pallas_call(
        _out_kernel,
        grid=(1,),
        in_specs=[
            pl.BlockSpec((h, nh, d), lambda i: (0, 0, 0)),
            pl.BlockSpec((h, nh, d), lambda i: (0, 0, 0)),
            pl.BlockSpec((h * d, dim), lambda i: (0, 0)),
            pl.BlockSpec((1, dim), lambda i: (0, 0)),
        ],
        out_specs=pl.BlockSpec((n, dim), lambda i: (0, 0)),
        out_shape=jax.ShapeDtypeStruct((n, dim), jnp.float32),
    )(attn_lo, attn_hi, wo2, bo2)

    return out.reshape(b, n, dim)


# speedup vs baseline: 1.2936x; 1.0285x over previous
import functools
import math

import jax
import jax.numpy as jnp
from jax.experimental import pallas as pl
from jax.experimental.pallas import tpu as pltpu

HEADS = 12
DIM_HEAD = 64
NUM_BUCKETS = 32
MAX_DISTANCE = 128
MASK_VALUE = -3.4028234663852886e38
MEM_SLOTS_PAD = 64


def _qkv_kernel(x_ref, w_ref, o_ref, *, scale):
    c = pl.program_id(0)
    sc = jnp.where(c < HEADS, scale, 1.0)
    o_ref[0] = (jnp.dot(x_ref[...], w_ref[0],
                        preferred_element_type=jnp.float32)
                * sc).astype(jnp.bfloat16)


def _out_kernel(a1_ref, a2_ref, a3_ref, a4_ref, w_ref, bias_ref, o_ref):
    nh = a1_ref.shape[1]
    for idx, ar in enumerate((a1_ref, a2_ref, a3_ref, a4_ref)):
        a = jnp.concatenate([ar[hh] for hh in range(HEADS)], axis=1)
        o_ref[idx * nh:(idx + 1) * nh, :] = (
            jnp.dot(a, w_ref[...], preferred_element_type=jnp.float32)
            + bias_ref[...])


def _bias_kernel(tab_ref, o_ref, *, bq):
    o_ref[...] = jnp.zeros_like(o_ref)
    sr = 128
    w = 3 * sr
    r = jax.lax.broadcasted_iota(jnp.int32, (sr, w), 0)
    cc = jax.lax.broadcasted_iota(jnp.int32, (sr, w), 1)
    delta = 2 * sr + r - cc
    npos = jnp.maximum(delta, 0)
    max_exact = NUM_BUCKETS // 2
    safe = jnp.maximum(npos, 1).astype(jnp.float32)
    val_large = max_exact + (
        jnp.log(safe * (1.0 / max_exact))
        * (max_exact / math.log(MAX_DISTANCE / max_exact))
    ).astype(jnp.int32)
    bucket = jnp.where(npos < max_exact, npos,
                       jnp.minimum(val_large, NUM_BUCKETS - 1))
    t31 = tab_ref[0, 0, NUM_BUCKETS - 1]
    adj = jnp.zeros((sr, w), jnp.float32)
    for t in range(NUM_BUCKETS - 1):
        adj = adj + jnp.where(bucket == t, tab_ref[0, 0, t] - t31, 0.0)
    adj = jnp.where(delta >= 0, adj, 0.0)
    for ri in range(bq // sr):
        cs = bq + (ri - 2) * sr
        o_ref[0, ri * sr:(ri + 1) * sr, cs:cs + w] = adj


def _attn_kernel(q_ref, k_ref, v_ref, w_ref, km_ref, vm_ref, tab_ref,
                 mask_ref, gate_ref, o_ref, s_ref, *, bq, n, scale, qoff):
    qi = pl.program_id(1) + qoff
    start = jnp.maximum(qi - 1, 0) * bq
    wstart = jnp.where(qi == 0, bq, 0)
    i = qi * bq + jax.lax.broadcasted_iota(jnp.int32, (bq, n), 0)
    j = jax.lax.broadcasted_iota(jnp.int32, (bq, n), 1)
    valid = j <= i

    for hh in range(2):
        q = q_ref[hh]
        k = k_ref[hh]
        v = v_ref[hh]

        s = jax.lax.dot_general(q, k, (((1,), (1,)), ((), ())),
                                preferred_element_type=jnp.float32)

        s_ref[hh] = s
        s_ref[hh, :, pl.ds(start, 2 * bq)] += \
            w_ref[hh, :, pl.ds(wstart, 2 * bq)]
        s = jnp.where(valid, s_ref[hh], MASK_VALUE)

        m = jnp.max(s, axis=1, keepdims=True)
        p = jnp.exp(s - m)
        l = jnp.sum(p, axis=1, keepdims=True)
        local = jnp.dot(p.astype(jnp.bfloat16), v,
                        preferred_element_type=jnp.float32) / l

        km = km_ref[hh]
        vm = vm_ref[hh]
        sm = jax.lax.dot_general(q, km, (((1,), (1,)), ((), ())),
                                 preferred_element_type=jnp.float32)
        mm = jnp.max(sm, axis=1, keepdims=True)
        pm = jnp.exp(sm - mm) * mask_ref[hh, :, 0:MEM_SLOTS_PAD]
        lm = jnp.sum(pm, axis=1, keepdims=True)
        mem = jnp.dot(pm.astype(jnp.bfloat16), vm,
                      preferred_element_type=jnp.float32) / lm

        g = jax.nn.sigmoid(gate_ref[hh, :, 0:1])
        o_ref[hh] = (local * g + mem * (1.0 - g)).astype(jnp.bfloat16)


def kernel(x, k_mem, v_mem, mem_mask, Wq, Wkv, Wo, bo, null_k, null_v,
           gate_param, rel_bias_table):
    b, n, dim = x.shape
    h, d = HEADS, DIM_HEAD
    topk = k_mem.shape[2]
    scale = d ** -0.5
    rel_scale = d ** 0.5
    nc = 3 * h

    x2 = x.reshape(n, dim).astype(jnp.bfloat16)
    w3 = (jnp.concatenate([Wq, Wkv], axis=1)
          .reshape(dim, nc, d).transpose(1, 0, 2)
          .astype(jnp.bfloat16))

    qkv = pl.pallas_call(
        functools.partial(_qkv_kernel, scale=scale),
        grid=(nc,),
        in_specs=[
            pl.BlockSpec((n, dim), lambda c: (0, 0)),
            pl.BlockSpec((1, dim, d), lambda c: (c, 0, 0)),
        ],
        out_specs=pl.BlockSpec((1, n, d), lambda c: (c, 0, 0)),
        out_shape=jax.ShapeDtypeStruct((nc, n, d), jnp.bfloat16),
    )(x2, w3)

    km = jnp.concatenate([null_k, k_mem[0]], axis=1)
    vm = jnp.concatenate([null_v, v_mem[0]], axis=1)
    pad = MEM_SLOTS_PAD - (1 + topk)
    km = jnp.pad(km, ((0, 0), (0, pad), (0, 0))).astype(jnp.bfloat16)
    vm = jnp.pad(vm, ((0, 0), (0, pad), (0, 0))).astype(jnp.bfloat16)
    maskf = jnp.concatenate(
        [jnp.ones((h, 1), jnp.float32), mem_mask[0].astype(jnp.float32),
         jnp.zeros((h, pad), jnp.float32)], axis=1)
    maskp = jnp.pad(maskf, ((0, 0), (0, 128 - MEM_SLOTS_PAD)))
    maskp = maskp.reshape(h, 1, 128)
    tabp = jnp.pad(rel_bias_table.T * rel_scale,
                   ((0, 0), (0, 128 - NUM_BUCKETS))).reshape(h, 1, 128)
    gatep = jnp.broadcast_to(gate_param.reshape(h, 1, 1),
                             (h, 1, 128)).astype(jnp.float32)

    bq = 256
    wtile = pl.pallas_call(
        functools.partial(_bias_kernel, bq=bq),
        grid=(h,),
        in_specs=[pl.BlockSpec((1, 1, 128), lambda hi: (hi, 0, 0))],
        out_specs=pl.BlockSpec((1, bq, 3 * bq), lambda hi: (hi, 0, 0)),
        out_shape=jax.ShapeDtypeStruct((h, bq, 3 * bq), jnp.float32),
    )(tabp)

    def attn_call(n_eff, qoff, nq):
        return pl.pallas_call(
            functools.partial(_attn_kernel, bq=bq, n=n_eff, scale=scale,
                              qoff=qoff),
            grid=(h // 2, nq),
            in_specs=[
                pl.BlockSpec((2, bq, d),
                             lambda hi, qi: (hi, qi + qoff, 0)),
                pl.BlockSpec((2, n_eff, d),
                             lambda hi, qi: (h // 2 + hi, 0, 0)),
                pl.BlockSpec((2, n_eff, d),
                             lambda hi, qi: (h + hi, 0, 0)),
                pl.BlockSpec((2, bq, 3 * bq), lambda hi, qi: (hi, 0, 0)),
                pl.BlockSpec((2, MEM_SLOTS_PAD, d),
                             lambda hi, qi: (hi, 0, 0)),
                pl.BlockSpec((2, MEM_SLOTS_PAD, d),
                             lambda hi, qi: (hi, 0, 0)),
                pl.BlockSpec((2, 1, 128), lambda hi, qi: (hi, 0, 0)),
                pl.BlockSpec((2, 1, 128), lambda hi, qi: (hi, 0, 0)),
                pl.BlockSpec((2, 1, 128), lambda hi, qi: (hi, 0, 0)),
            ],
            out_specs=pl.BlockSpec((2, bq, d), lambda hi, qi: (hi, qi, 0)),
            out_shape=jax.ShapeDtypeStruct((h, nq * bq, d), jnp.bfloat16),
            scratch_shapes=[pltpu.VMEM((2, bq, n_eff), jnp.float32)],
        )(qkv, qkv, qkv, wtile, km, vm, tabp, maskp, gatep)

    nh = n // 4
    attn_parts = [attn_call((qu + 1) * nh, qu * (nh // bq), nh // bq)
                  for qu in range(4)]

    wo2 = Wo.astype(jnp.bfloat16)
    bo2 = bo.reshape(1, dim)
    out = pl.pallas_call(
        _out_kernel,
        grid=(1,),
        in_specs=[
            pl.BlockSpec((h, nh, d), lambda i: (0, 0, 0)),
            pl.BlockSpec((h, nh, d), lambda i: (0, 0, 0)),
            pl.BlockSpec((h, nh, d), lambda i: (0, 0, 0)),
            pl.BlockSpec((h, nh, d), lambda i: (0, 0, 0)),
            pl.BlockSpec((h * d, dim), lambda i: (0, 0)),
            pl.BlockSpec((1, dim), lambda i: (0, 0)),
        ],
        out_specs=pl.BlockSpec((n, dim), lambda i: (0, 0)),
        out_shape=jax.ShapeDtypeStruct((n, dim), jnp.float32),
    )(*attn_parts, wo2, bo2)

    return out.reshape(b, n, dim)


# speedup vs baseline: 1.5045x; 1.1630x over previous
import functools
import math

import jax
import jax.numpy as jnp
from jax.experimental import pallas as pl
from jax.experimental.pallas import tpu as pltpu

HEADS = 12
DIM_HEAD = 64
NUM_BUCKETS = 32
MAX_DISTANCE = 128
MASK_VALUE = -3.4028234663852886e38
MEM_SLOTS_PAD = 64


def _proj_kernel(x_ref, w_ref, o_ref, *, scale):
    xb = x_ref[...].astype(jnp.bfloat16)
    wb = w_ref[...].astype(jnp.bfloat16)
    oo = jnp.dot(xb, wb, preferred_element_type=jnp.float32)
    if scale != 1.0:
        oo = oo * scale
    o_ref[0] = oo[:, 0:DIM_HEAD].astype(jnp.bfloat16)
    o_ref[1] = oo[:, DIM_HEAD:].astype(jnp.bfloat16)


def _out_kernel(a1_ref, a2_ref, a3_ref, a4_ref, w_ref, bias_ref, o_ref):
    nh = a1_ref.shape[1]
    wb = w_ref[...].astype(jnp.bfloat16)
    for idx, ar in enumerate((a1_ref, a2_ref, a3_ref, a4_ref)):
        a = jnp.concatenate([ar[hh] for hh in range(HEADS)], axis=1)
        o_ref[idx * nh:(idx + 1) * nh, :] = (
            jnp.dot(a, wb, preferred_element_type=jnp.float32)
            + bias_ref[...])


def _bias_kernel(tab_ref, o_ref, *, bq):
    o_ref[...] = jnp.zeros_like(o_ref)
    sr = 128
    w = 3 * sr
    r = jax.lax.broadcasted_iota(jnp.int32, (sr, w), 0)
    cc = jax.lax.broadcasted_iota(jnp.int32, (sr, w), 1)
    delta = 2 * sr + r - cc
    npos = jnp.maximum(delta, 0)
    max_exact = NUM_BUCKETS // 2
    safe = jnp.maximum(npos, 1).astype(jnp.float32)
    val_large = max_exact + (
        jnp.log(safe * (1.0 / max_exact))
        * (max_exact / math.log(MAX_DISTANCE / max_exact))
    ).astype(jnp.int32)
    bucket = jnp.where(npos < max_exact, npos,
                       jnp.minimum(val_large, NUM_BUCKETS - 1))
    t31 = tab_ref[0, 0, NUM_BUCKETS - 1]
    adj = jnp.zeros((sr, w), jnp.float32)
    for t in range(NUM_BUCKETS - 1):
        adj = adj + jnp.where(bucket == t, tab_ref[0, 0, t] - t31, 0.0)
    adj = jnp.where(delta >= 0, adj, 0.0)
    for ri in range(bq // sr):
        cs = bq + (ri - 2) * sr
        o_ref[0, ri * sr:(ri + 1) * sr, cs:cs + w] = adj


def _attn_kernel(q_ref, k_ref, v_ref, w_ref, km_ref, vm_ref, tab_ref,
                 mask_ref, gate_ref, o_ref, s_ref, *, bq, n, scale, qoff):
    qi = pl.program_id(1) + qoff
    start = jnp.maximum(qi - 1, 0) * bq
    wstart = jnp.where(qi == 0, bq, 0)
    i = qi * bq + jax.lax.broadcasted_iota(jnp.int32, (bq, n), 0)
    j = jax.lax.broadcasted_iota(jnp.int32, (bq, n), 1)
    valid = j <= i

    for hh in range(2):
        q = q_ref[hh]
        k = k_ref[hh]
        v = v_ref[hh]

        s = jax.lax.dot_general(q, k, (((1,), (1,)), ((), ())),
                                preferred_element_type=jnp.float32)

        s_ref[hh] = s
        s_ref[hh, :, pl.ds(start, 2 * bq)] += \
            w_ref[hh, :, pl.ds(wstart, 2 * bq)]
        s = jnp.where(valid, s_ref[hh], MASK_VALUE)

        m = jnp.max(s, axis=1, keepdims=True)
        p = jnp.exp(s - m)
        l = jnp.sum(p, axis=1, keepdims=True)
        local = jnp.dot(p.astype(jnp.bfloat16), v,
                        preferred_element_type=jnp.float32) / l

        km = km_ref[hh]
        vm = vm_ref[hh]
        sm = jax.lax.dot_general(q, km, (((1,), (1,)), ((), ())),
                                 preferred_element_type=jnp.float32)
        mm = jnp.max(sm, axis=1, keepdims=True)
        pm = jnp.exp(sm - mm) * mask_ref[hh, :, 0:MEM_SLOTS_PAD]
        lm = jnp.sum(pm, axis=1, keepdims=True)
        mem = jnp.dot(pm.astype(jnp.bfloat16), vm,
                      preferred_element_type=jnp.float32) / lm

        g = jax.nn.sigmoid(gate_ref[hh, :, 0:1])
        o_ref[hh] = (local * g + mem * (1.0 - g)).astype(jnp.bfloat16)


def kernel(x, k_mem, v_mem, mem_mask, Wq, Wkv, Wo, bo, null_k, null_v,
           gate_param, rel_bias_table):
    b, n, dim = x.shape
    h, d = HEADS, DIM_HEAD
    topk = k_mem.shape[2]
    scale = d ** -0.5
    rel_scale = d ** 0.5
    nc = 3 * h

    x2 = x.reshape(n, dim)

    qarr = pl.pallas_call(
        functools.partial(_proj_kernel, scale=scale),
        grid=(h // 2,),
        in_specs=[
            pl.BlockSpec((n, dim), lambda c: (0, 0)),
            pl.BlockSpec((dim, 2 * d), lambda c: (0, c)),
        ],
        out_specs=pl.BlockSpec((2, n, d), lambda c: (c, 0, 0)),
        out_shape=jax.ShapeDtypeStruct((h, n, d), jnp.bfloat16),
    )(x2, Wq)

    kvarr = pl.pallas_call(
        functools.partial(_proj_kernel, scale=1.0),
        grid=(h,),
        in_specs=[
            pl.BlockSpec((n, dim), lambda c: (0, 0)),
            pl.BlockSpec((dim, 2 * d), lambda c: (0, c)),
        ],
        out_specs=pl.BlockSpec((2, n, d), lambda c: (c, 0, 0)),
        out_shape=jax.ShapeDtypeStruct((2 * h, n, d), jnp.bfloat16),
    )(x2, Wkv)

    km = jnp.concatenate([null_k, k_mem[0]], axis=1)
    vm = jnp.concatenate([null_v, v_mem[0]], axis=1)
    pad = MEM_SLOTS_PAD - (1 + topk)
    km = jnp.pad(km, ((0, 0), (0, pad), (0, 0))).astype(jnp.bfloat16)
    vm = jnp.pad(vm, ((0, 0), (0, pad), (0, 0))).astype(jnp.bfloat16)
    maskf = jnp.concatenate(
        [jnp.ones((h, 1), jnp.float32), mem_mask[0].astype(jnp.float32),
         jnp.zeros((h, pad), jnp.float32)], axis=1)
    maskp = jnp.pad(maskf, ((0, 0), (0, 128 - MEM_SLOTS_PAD)))
    maskp = maskp.reshape(h, 1, 128)
    tabp = jnp.pad(rel_bias_table.T * rel_scale,
                   ((0, 0), (0, 128 - NUM_BUCKETS))).reshape(h, 1, 128)
    gatep = jnp.broadcast_to(gate_param.reshape(h, 1, 1),
                             (h, 1, 128)).astype(jnp.float32)

    bq = 256
    wtile = pl.pallas_call(
        functools.partial(_bias_kernel, bq=bq),
        grid=(h,),
        in_specs=[pl.BlockSpec((1, 1, 128), lambda hi: (hi, 0, 0))],
        out_specs=pl.BlockSpec((1, bq, 3 * bq), lambda hi: (hi, 0, 0)),
        out_shape=jax.ShapeDtypeStruct((h, bq, 3 * bq), jnp.float32),
    )(tabp)

    def attn_call(n_eff, qoff, nq):
        return pl.pallas_call(
            functools.partial(_attn_kernel, bq=bq, n=n_eff, scale=scale,
                              qoff=qoff),
            grid=(h // 2, nq),
            in_specs=[
                pl.BlockSpec((2, bq, d),
                             lambda hi, qi: (hi, qi + qoff, 0)),
                pl.BlockSpec((2, n_eff, d), lambda hi, qi: (hi, 0, 0)),
                pl.BlockSpec((2, n_eff, d),
                             lambda hi, qi: (h // 2 + hi, 0, 0)),
                pl.BlockSpec((2, bq, 3 * bq), lambda hi, qi: (hi, 0, 0)),
                pl.BlockSpec((2, MEM_SLOTS_PAD, d),
                             lambda hi, qi: (hi, 0, 0)),
                pl.BlockSpec((2, MEM_SLOTS_PAD, d),
                             lambda hi, qi: (hi, 0, 0)),
                pl.BlockSpec((2, 1, 128), lambda hi, qi: (hi, 0, 0)),
                pl.BlockSpec((2, 1, 128), lambda hi, qi: (hi, 0, 0)),
                pl.BlockSpec((2, 1, 128), lambda hi, qi: (hi, 0, 0)),
            ],
            out_specs=pl.BlockSpec((2, bq, d), lambda hi, qi: (hi, qi, 0)),
            out_shape=jax.ShapeDtypeStruct((h, nq * bq, d), jnp.bfloat16),
            scratch_shapes=[pltpu.VMEM((2, bq, n_eff), jnp.float32)],
        )(qarr, kvarr, kvarr, wtile, km, vm, tabp, maskp, gatep)

    nh = n // 4
    attn_parts = [attn_call((qu + 1) * nh, qu * (nh // bq), nh // bq)
                  for qu in range(4)]

    bo2 = bo.reshape(1, dim)
    out = pl.pallas_call(
        _out_kernel,
        grid=(1,),
        in_specs=[
            pl.BlockSpec((h, nh, d), lambda i: (0, 0, 0)),
            pl.BlockSpec((h, nh, d), lambda i: (0, 0, 0)),
            pl.BlockSpec((h, nh, d), lambda i: (0, 0, 0)),
            pl.BlockSpec((h, nh, d), lambda i: (0, 0, 0)),
            pl.BlockSpec((h * d, dim), lambda i: (0, 0)),
            pl.BlockSpec((1, dim), lambda i: (0, 0)),
        ],
        out_specs=pl.BlockSpec((n, dim), lambda i: (0, 0)),
        out_shape=jax.ShapeDtypeStruct((n, dim), jnp.float32),
    )(*attn_parts, Wo, bo2)

    return out.reshape(b, n, dim)


# speedup vs baseline: 1.5358x; 1.0208x over previous
import functools
import math

import jax
import jax.numpy as jnp
from jax.experimental import pallas as pl
from jax.experimental.pallas import tpu as pltpu

HEADS = 12
DIM_HEAD = 64
NUM_BUCKETS = 32
MAX_DISTANCE = 128
MASK_VALUE = -3.4028234663852886e38
MEM_SLOTS_PAD = 64


def _proj_kernel(x_ref, w_ref, o_ref, *, scale):
    xb = x_ref[...].astype(jnp.bfloat16)
    wb = w_ref[...].astype(jnp.bfloat16)
    oo = jnp.dot(xb, wb, preferred_element_type=jnp.float32)
    if scale != 1.0:
        oo = oo * scale
    o_ref[0] = oo[:, 0:DIM_HEAD].astype(jnp.bfloat16)
    o_ref[1] = oo[:, DIM_HEAD:].astype(jnp.bfloat16)


def _out_kernel(a1_ref, a2_ref, a3_ref, a4_ref, w_ref, bias_ref, o_ref):
    nh = a1_ref.shape[1]
    wb = w_ref[...].astype(jnp.bfloat16)
    for idx, ar in enumerate((a1_ref, a2_ref, a3_ref, a4_ref)):
        a = jnp.concatenate([ar[hh] for hh in range(HEADS)], axis=1)
        o_ref[idx * nh:(idx + 1) * nh, :] = (
            jnp.dot(a, wb, preferred_element_type=jnp.float32)
            + bias_ref[...])


def _bias_kernel(tab_ref, o_ref, *, bq):
    r0 = jax.lax.broadcasted_iota(jnp.int32, o_ref.shape[1:], 0)
    c0 = jax.lax.broadcasted_iota(jnp.int32, o_ref.shape[1:], 1)
    o_ref[...] = jnp.where(c0 > bq + r0, -1e38, 0.0)[None]
    sr = 128
    w = 3 * sr
    r = jax.lax.broadcasted_iota(jnp.int32, (sr, w), 0)
    cc = jax.lax.broadcasted_iota(jnp.int32, (sr, w), 1)
    delta = 2 * sr + r - cc
    npos = jnp.maximum(delta, 0)
    max_exact = NUM_BUCKETS // 2
    safe = jnp.maximum(npos, 1).astype(jnp.float32)
    val_large = max_exact + (
        jnp.log(safe * (1.0 / max_exact))
        * (max_exact / math.log(MAX_DISTANCE / max_exact))
    ).astype(jnp.int32)
    bucket = jnp.where(npos < max_exact, npos,
                       jnp.minimum(val_large, NUM_BUCKETS - 1))
    t31 = tab_ref[0, 0, NUM_BUCKETS - 1]
    adj = jnp.zeros((sr, w), jnp.float32)
    for t in range(NUM_BUCKETS - 1):
        adj = adj + jnp.where(bucket == t, tab_ref[0, 0, t] - t31, 0.0)
    adj = jnp.where(delta >= 0, adj, -1e38)
    for ri in range(bq // sr):
        cs = bq + (ri - 2) * sr
        o_ref[0, ri * sr:(ri + 1) * sr, cs:cs + w] = adj


def _attn_kernel(q_ref, k_ref, v_ref, w_ref, km_ref, vm_ref, tab_ref,
                 mask_ref, gate_ref, o_ref, s_ref, *, bq, n, scale, qoff):
    qi = pl.program_id(1) + qoff
    start = jnp.maximum(qi - 1, 0) * bq
    wstart = jnp.where(qi == 0, bq, 0)

    for hh in range(2):
        q = q_ref[hh]
        k = k_ref[hh]
        v = v_ref[hh]

        s = jax.lax.dot_general(q, k, (((1,), (1,)), ((), ())),
                                preferred_element_type=jnp.float32)

        s_ref[hh] = s
        s_ref[hh, :, pl.ds(start, 2 * bq)] += \
            w_ref[hh, :, pl.ds(wstart, 2 * bq)]

        @pl.when(qi % 2 == 0)
        def _():
            s_ref[hh, :, n - bq:] = jnp.full((bq, bq), MASK_VALUE,
                                             jnp.float32)

        s = s_ref[hh]

        m = jnp.max(s, axis=1, keepdims=True)
        p = jnp.exp(s - m)
        l = jnp.sum(p, axis=1, keepdims=True)
        local = jnp.dot(p.astype(jnp.bfloat16), v,
                        preferred_element_type=jnp.float32) / l

        km = km_ref[hh]
        vm = vm_ref[hh]
        sm = jax.lax.dot_general(q, km, (((1,), (1,)), ((), ())),
                                 preferred_element_type=jnp.float32)
        mm = jnp.max(sm, axis=1, keepdims=True)
        pm = jnp.exp(sm - mm) * mask_ref[hh, :, 0:MEM_SLOTS_PAD]
        lm = jnp.sum(pm, axis=1, keepdims=True)
        mem = jnp.dot(pm.astype(jnp.bfloat16), vm,
                      preferred_element_type=jnp.float32) / lm

        g = jax.nn.sigmoid(gate_ref[hh, :, 0:1])
        o_ref[hh] = (local * g + mem * (1.0 - g)).astype(jnp.bfloat16)


def kernel(x, k_mem, v_mem, mem_mask, Wq, Wkv, Wo, bo, null_k, null_v,
           gate_param, rel_bias_table):
    b, n, dim = x.shape
    h, d = HEADS, DIM_HEAD
    topk = k_mem.shape[2]
    scale = d ** -0.5
    rel_scale = d ** 0.5
    nc = 3 * h

    x2 = x.reshape(n, dim)

    qarr = pl.pallas_call(
        functools.partial(_proj_kernel, scale=scale),
        grid=(h // 2,),
        in_specs=[
            pl.BlockSpec((n, dim), lambda c: (0, 0)),
            pl.BlockSpec((dim, 2 * d), lambda c: (0, c)),
        ],
        out_specs=pl.BlockSpec((2, n, d), lambda c: (c, 0, 0)),
        out_shape=jax.ShapeDtypeStruct((h, n, d), jnp.bfloat16),
    )(x2, Wq)

    kvarr = pl.pallas_call(
        functools.partial(_proj_kernel, scale=1.0),
        grid=(h,),
        in_specs=[
            pl.BlockSpec((n, dim), lambda c: (0, 0)),
            pl.BlockSpec((dim, 2 * d), lambda c: (0, c)),
        ],
        out_specs=pl.BlockSpec((2, n, d), lambda c: (c, 0, 0)),
        out_shape=jax.ShapeDtypeStruct((2 * h, n, d), jnp.bfloat16),
    )(x2, Wkv)

    km = jnp.concatenate([null_k, k_mem[0]], axis=1)
    vm = jnp.concatenate([null_v, v_mem[0]], axis=1)
    pad = MEM_SLOTS_PAD - (1 + topk)
    km = jnp.pad(km, ((0, 0), (0, pad), (0, 0))).astype(jnp.bfloat16)
    vm = jnp.pad(vm, ((0, 0), (0, pad), (0, 0))).astype(jnp.bfloat16)
    maskf = jnp.concatenate(
        [jnp.ones((h, 1), jnp.float32), mem_mask[0].astype(jnp.float32),
         jnp.zeros((h, pad), jnp.float32)], axis=1)
    maskp = jnp.pad(maskf, ((0, 0), (0, 128 - MEM_SLOTS_PAD)))
    maskp = maskp.reshape(h, 1, 128)
    tabp = jnp.pad(rel_bias_table.T * rel_scale,
                   ((0, 0), (0, 128 - NUM_BUCKETS))).reshape(h, 1, 128)
    gatep = jnp.broadcast_to(gate_param.reshape(h, 1, 1),
                             (h, 1, 128)).astype(jnp.float32)

    bq = 256
    wtile = pl.pallas_call(
        functools.partial(_bias_kernel, bq=bq),
        grid=(h,),
        in_specs=[pl.BlockSpec((1, 1, 128), lambda hi: (hi, 0, 0))],
        out_specs=pl.BlockSpec((1, bq, 3 * bq), lambda hi: (hi, 0, 0)),
        out_shape=jax.ShapeDtypeStruct((h, bq, 3 * bq), jnp.float32),
    )(tabp)

    def attn_call(n_eff, qoff, nq):
        return pl.pallas_call(
            functools.partial(_attn_kernel, bq=bq, n=n_eff, scale=scale,
                              qoff=qoff),
            grid=(h // 2, nq),
            in_specs=[
                pl.BlockSpec((2, bq, d),
                             lambda hi, qi: (hi, qi + qoff, 0)),
                pl.BlockSpec((2, n_eff, d), lambda hi, qi: (hi, 0, 0)),
                pl.BlockSpec((2, n_eff, d),
                             lambda hi, qi: (h // 2 + hi, 0, 0)),
                pl.BlockSpec((2, bq, 3 * bq), lambda hi, qi: (hi, 0, 0)),
                pl.BlockSpec((2, MEM_SLOTS_PAD, d),
                             lambda hi, qi: (hi, 0, 0)),
                pl.BlockSpec((2, MEM_SLOTS_PAD, d),
                             lambda hi, qi: (hi, 0, 0)),
                pl.BlockSpec((2, 1, 128), lambda hi, qi: (hi, 0, 0)),
                pl.BlockSpec((2, 1, 128), lambda hi, qi: (hi, 0, 0)),
                pl.BlockSpec((2, 1, 128), lambda hi, qi: (hi, 0, 0)),
            ],
            out_specs=pl.BlockSpec((2, bq, d), lambda hi, qi: (hi, qi, 0)),
            out_shape=jax.ShapeDtypeStruct((h, nq * bq, d), jnp.bfloat16),
            scratch_shapes=[pltpu.VMEM((2, bq, n_eff), jnp.float32)],
        )(qarr, kvarr, kvarr, wtile, km, vm, tabp, maskp, gatep)

    nh = n // 4
    attn_parts = [attn_call((qu + 1) * nh, qu * (nh // bq), nh // bq)
                  for qu in range(4)]

    bo2 = bo.reshape(1, dim)
    out = pl.pallas_call(
        _out_kernel,
        grid=(1,),
        in_specs=[
            pl.BlockSpec((h, nh, d), lambda i: (0, 0, 0)),
            pl.BlockSpec((h, nh, d), lambda i: (0, 0, 0)),
            pl.BlockSpec((h, nh, d), lambda i: (0, 0, 0)),
            pl.BlockSpec((h, nh, d), lambda i: (0, 0, 0)),
            pl.BlockSpec((h * d, dim), lambda i: (0, 0)),
            pl.BlockSpec((1, dim), lambda i: (0, 0)),
        ],
        out_specs=pl.BlockSpec((n, dim), lambda i: (0, 0)),
        out_shape=jax.ShapeDtypeStruct((n, dim), jnp.float32),
    )(*attn_parts, Wo, bo2)

    return out.reshape(b, n, dim)
